# Initial kernel scaffold; baseline (speedup 1.0000x reference)
import functools

import jax
import jax.numpy as jnp
from jax import lax
from jax.experimental import pallas as pl
from jax.experimental.pallas import tpu as pltpu
from jax.experimental.pallas import tpu_sc as plsc

E = 320000
D = 128
NH = 10000
NG = 500
NW = 32
EPT = E // NW
C = 80
NCHUNK = EPT // C
L = 16


def _fast_sqrt(v):
    xi = lax.bitcast_convert_type(v, jnp.int32)
    yi = jnp.int32(0x5F3759DF) - lax.shift_right_logical(xi, 1)
    y = lax.bitcast_convert_type(yi, jnp.float32)
    for _ in range(3):
        y = y * (jnp.float32(1.5) - jnp.float32(0.5) * v * y * y)
    return jnp.where(v > 0.0, v * y, jnp.float32(0.0))


def _make_sc_kernel():
    mesh = plsc.VectorSubcoreMesh(core_axis_name="c", subcore_axis_name="s")

    @functools.partial(
        pl.kernel,
        mesh=mesh,
        out_type=jax.ShapeDtypeStruct((E,), jnp.float32),
        scratch_types=[
            pltpu.VMEM((NG, D), jnp.float32),
            pltpu.VMEM((2, C, D), jnp.float32),
            pltpu.VMEM((2, C, D), jnp.float32),
            pltpu.VMEM((2, C), jnp.int32),
            pltpu.VMEM((2, C), jnp.int32),
            pltpu.VMEM((2, C), jnp.int32),
            pltpu.VMEM((EPT,), jnp.float32),
            pltpu.SemaphoreType.DMA,
            pltpu.SemaphoreType.DMA,
            pltpu.SemaphoreType.DMA,
            pltpu.SemaphoreType.DMA,
            pltpu.SemaphoreType.DMA,
            pltpu.SemaphoreType.DMA,
        ],
    )
    def sc_kernel(h_hbm, g_hbm, row_hbm, col_hbm, et_hbm, out_hbm,
                  g_buf, r_buf, c_buf, ri_buf, ci_buf, t_buf, o_buf,
                  gsem0, gsem1, isem0, isem1, tsem0, tsem1):
        gsem = (gsem0, gsem1)
        isem = (isem0, isem1)
        tsem = (tsem0, tsem1)
        wid = lax.axis_index("s") * 2 + lax.axis_index("c")
        base = wid * EPT

        pltpu.sync_copy(g_hbm, g_buf)

        def start_gathers(b):
            pltpu.async_copy(h_hbm.at[ri_buf.at[b]], r_buf.at[b], gsem[b])
            pltpu.async_copy(h_hbm.at[ci_buf.at[b]], c_buf.at[b], gsem[b])

        for b in range(2):
            off = base + b * C
            pltpu.sync_copy(row_hbm.at[pl.ds(off, C)], ri_buf.at[b])
            pltpu.sync_copy(col_hbm.at[pl.ds(off, C)], ci_buf.at[b])
            pltpu.async_copy(et_hbm.at[pl.ds(off, C)], t_buf.at[b], tsem[b])
            start_gathers(b)

        def process(k, b):
            pltpu.make_async_copy(h_hbm.at[pl.ds(0, C)], r_buf.at[b], gsem[b]).wait()
            pltpu.make_async_copy(h_hbm.at[pl.ds(0, C)], c_buf.at[b], gsem[b]).wait()

            have_next = k + 2 < NCHUNK
            off2 = base + (k + 2) * C

            @pl.when(have_next)
            def _():
                pltpu.async_copy(row_hbm.at[pl.ds(off2, C)], ri_buf.at[b], isem[b])
                pltpu.async_copy(col_hbm.at[pl.ds(off2, C)], ci_buf.at[b], isem[b])

            pltpu.make_async_copy(et_hbm.at[pl.ds(0, C)], t_buf.at[b], tsem[b]).wait()

            koff = k * C

            def edge_body(e, carry):
                t = t_buf[b, e]
                acc = jnp.zeros((L,), jnp.float32)
                for j in range(D // L):
                    sl = pl.ds(j * L, L)
                    rv = r_buf[b, e, sl]
                    cv = c_buf[b, e, sl]
                    gv = g_buf[t, sl]
                    diff = rv * gv - cv
                    acc = acc + diff * diff
                o_buf[koff + e] = jnp.sum(acc)
                return carry

            lax.fori_loop(0, C, edge_body, 0)

            @pl.when(have_next)
            def _():
                pltpu.make_async_copy(row_hbm.at[pl.ds(0, C)], ri_buf.at[b], isem[b]).wait()
                pltpu.make_async_copy(col_hbm.at[pl.ds(0, C)], ci_buf.at[b], isem[b]).wait()
                start_gathers(b)
                pltpu.async_copy(et_hbm.at[pl.ds(off2, C)], t_buf.at[b], tsem[b])

        def pair_body(i, carry):
            process(2 * i, 0)
            process(2 * i + 1, 1)
            return carry

        lax.fori_loop(0, NCHUNK // 2, pair_body, 0)
        process(NCHUNK - 1, (NCHUNK - 1) % 2)

        def sqrt_body(i, carry):
            sl = pl.ds(i * L, L)
            o_buf[sl] = _fast_sqrt(o_buf[sl])
            return carry

        lax.fori_loop(0, EPT // L, sqrt_body, 0)

        pltpu.sync_copy(o_buf, out_hbm.at[pl.ds(base, EPT)])

    return sc_kernel


_SC_KERNEL = _make_sc_kernel()


@jax.jit
def kernel(h, g, edge_idx, edge_type):
    row = edge_idx[0].astype(jnp.int32)
    col = edge_idx[1].astype(jnp.int32)
    et = edge_type.astype(jnp.int32)
    return _SC_KERNEL(h, g, row, col, et)


# speedup vs baseline: 1.3172x; 1.3172x over previous
import functools

import jax
import jax.numpy as jnp
from jax import lax
from jax.experimental import pallas as pl
from jax.experimental.pallas import tpu as pltpu
from jax.experimental.pallas import tpu_sc as plsc

E = 320000
D = 128
NH = 10000
NG = 500
NW = 32
EPT = E // NW
C = 80
NCHUNK = EPT // C
L = 16


def _fast_sqrt(v):
    xi = lax.bitcast_convert_type(v, jnp.int32)
    yi = jnp.int32(0x5F3759DF) - lax.shift_right_logical(xi, 1)
    y = lax.bitcast_convert_type(yi, jnp.float32)
    for _ in range(3):
        y = y * (jnp.float32(1.5) - jnp.float32(0.5) * v * y * y)
    return jnp.where(v > 0.0, v * y, jnp.float32(0.0))


def _make_sc_kernel():
    mesh = plsc.VectorSubcoreMesh(core_axis_name="c", subcore_axis_name="s")

    @functools.partial(
        pl.kernel,
        mesh=mesh,
        out_type=jax.ShapeDtypeStruct((E,), jnp.float32),
        compiler_params=pltpu.CompilerParams(needs_layout_passes=False),
        scratch_types=[
            pltpu.VMEM((NG, D), jnp.float32),
            pltpu.VMEM((2, C, D), jnp.float32),
            pltpu.VMEM((2, C, D), jnp.float32),
            pltpu.VMEM((2, C), jnp.int32),
            pltpu.VMEM((2, C), jnp.int32),
            pltpu.VMEM((2, C), jnp.int32),
            pltpu.VMEM((EPT,), jnp.float32),
            pltpu.SemaphoreType.DMA,
            pltpu.SemaphoreType.DMA,
            pltpu.SemaphoreType.DMA,
            pltpu.SemaphoreType.DMA,
            pltpu.SemaphoreType.DMA,
            pltpu.SemaphoreType.DMA,
        ],
    )
    def sc_kernel(h_hbm, g_hbm, row_hbm, col_hbm, et_hbm, out_hbm,
                  g_buf, r_buf, c_buf, ri_buf, ci_buf, t_buf, o_buf,
                  gsem0, gsem1, isem0, isem1, tsem0, tsem1):
        gsem = (gsem0, gsem1)
        isem = (isem0, isem1)
        tsem = (tsem0, tsem1)
        wid = lax.axis_index("s") * 2 + lax.axis_index("c")
        base = wid * EPT

        pltpu.sync_copy(g_hbm, g_buf)

        def start_gathers(b):
            pltpu.async_copy(h_hbm.at[ri_buf.at[b]], r_buf.at[b], gsem[b])
            pltpu.async_copy(h_hbm.at[ci_buf.at[b]], c_buf.at[b], gsem[b])

        for b in range(2):
            off = base + b * C
            pltpu.sync_copy(row_hbm.at[pl.ds(off, C)], ri_buf.at[b])
            pltpu.sync_copy(col_hbm.at[pl.ds(off, C)], ci_buf.at[b])
            pltpu.async_copy(et_hbm.at[pl.ds(off, C)], t_buf.at[b], tsem[b])
            start_gathers(b)

        def process(k, b):
            pltpu.make_async_copy(h_hbm.at[pl.ds(0, C)], r_buf.at[b], gsem[b]).wait()
            pltpu.make_async_copy(h_hbm.at[pl.ds(0, C)], c_buf.at[b], gsem[b]).wait()

            have_next = k + 2 < NCHUNK
            off2 = base + (k + 2) * C

            @pl.when(have_next)
            def _():
                pltpu.async_copy(row_hbm.at[pl.ds(off2, C)], ri_buf.at[b], isem[b])
                pltpu.async_copy(col_hbm.at[pl.ds(off2, C)], ci_buf.at[b], isem[b])

            pltpu.make_async_copy(et_hbm.at[pl.ds(0, C)], t_buf.at[b], tsem[b]).wait()

            koff = k * C

            ev = lax.iota(jnp.int32, L)
            bsplat = jnp.full((L,), b, jnp.int32)
            for q in range(C // L):
                esplat = ev + jnp.int32(q * L)
                tv = t_buf[b, pl.ds(q * L, L)]

                def d_body(d, acc):
                    dsplat = jnp.full((L,), d, jnp.int32)
                    rv = plsc.load_gather(r_buf, [bsplat, esplat, dsplat])
                    cv = plsc.load_gather(c_buf, [bsplat, esplat, dsplat])
                    gv = plsc.load_gather(g_buf, [tv, dsplat])
                    diff = rv * gv - cv
                    return acc + diff * diff

                acc = lax.fori_loop(0, D, d_body,
                                    jnp.zeros((L,), jnp.float32), unroll=4)
                o_buf[pl.ds(koff + q * L, L)] = acc

            @pl.when(have_next)
            def _():
                pltpu.make_async_copy(row_hbm.at[pl.ds(0, C)], ri_buf.at[b], isem[b]).wait()
                pltpu.make_async_copy(col_hbm.at[pl.ds(0, C)], ci_buf.at[b], isem[b]).wait()
                start_gathers(b)
                pltpu.async_copy(et_hbm.at[pl.ds(off2, C)], t_buf.at[b], tsem[b])

        def pair_body(i, carry):
            process(2 * i, 0)
            process(2 * i + 1, 1)
            return carry

        lax.fori_loop(0, NCHUNK // 2, pair_body, 0)
        process(NCHUNK - 1, (NCHUNK - 1) % 2)

        def sqrt_body(i, carry):
            sl = pl.ds(i * L, L)
            o_buf[sl] = _fast_sqrt(o_buf[sl])
            return carry

        lax.fori_loop(0, EPT // L, sqrt_body, 0)

        pltpu.sync_copy(o_buf, out_hbm.at[pl.ds(base, EPT)])

    return sc_kernel


_SC_KERNEL = _make_sc_kernel()


@jax.jit
def kernel(h, g, edge_idx, edge_type):
    row = edge_idx[0].astype(jnp.int32)
    col = edge_idx[1].astype(jnp.int32)
    et = edge_type.astype(jnp.int32)
    return _SC_KERNEL(h, g, row, col, et)
